# Initial kernel scaffold; baseline (speedup 1.0000x reference)
import jax
import jax.numpy as jnp
from jax.experimental import pallas as pl


def kernel(x, embed):
    raise NotImplementedError("write your pallas kernel here")


# speedup vs baseline: 1.0953x; 1.0953x over previous
import functools

import jax
import jax.numpy as jnp
from jax import lax
from jax.experimental import pallas as pl
from jax.experimental.pallas import tpu as pltpu
from jax.experimental.pallas import tpu_sc as plsc

EMBED_DIM = 32
NUM_CORES = 2
NUM_SUBCORES = 16
NUM_WORKERS = NUM_CORES * NUM_SUBCORES
CHUNK = 1024


@functools.partial(jax.jit, static_argnames=("b_per_w", "n_chunks"))
def _emb_lookup(x_flat, embed, *, b_per_w, n_chunks):
    B = x_flat.shape[0]
    mesh = plsc.VectorSubcoreMesh(core_axis_name="c", subcore_axis_name="s")

    @functools.partial(
        pl.kernel,
        mesh=mesh,
        out_type=jax.ShapeDtypeStruct((B, EMBED_DIM), jnp.float32),
        scratch_types=[
            pltpu.VMEM((CHUNK,), jnp.int32),
            pltpu.VMEM((CHUNK, EMBED_DIM), jnp.float32),
            pltpu.SemaphoreType.DMA,
        ],
        compiler_params=pltpu.CompilerParams(use_tc_tiling_on_sc=False),
    )
    def emb_kernel(idx_hbm, table_hbm, out_hbm, idx_v, rows_v, sem):
        wid = lax.axis_index("s") * NUM_CORES + lax.axis_index("c")
        base = wid * b_per_w

        def body(g, carry):
            off = base + g * CHUNK
            pltpu.sync_copy(idx_hbm.at[pl.ds(off, CHUNK)], idx_v)
            pltpu.async_copy(table_hbm.at[idx_v], rows_v, sem).wait()
            pltpu.sync_copy(rows_v, out_hbm.at[pl.ds(off, CHUNK)])
            return carry

        lax.fori_loop(0, n_chunks, body, 0)

    return emb_kernel(x_flat, embed)


def kernel(x, embed):
    B = x.shape[0] * x.shape[1]
    b_per_w = B // NUM_WORKERS
    n_chunks = b_per_w // CHUNK
    assert b_per_w % CHUNK == 0
    x_flat = x.reshape(B)
    out = _emb_lookup(x_flat, embed, b_per_w=b_per_w, n_chunks=n_chunks)
    return out.reshape(x.shape[0], x.shape[1], EMBED_DIM)


# speedup vs baseline: 1.1104x; 1.0139x over previous
import functools

import jax
import jax.numpy as jnp
from jax import lax
from jax.experimental import pallas as pl
from jax.experimental.pallas import tpu as pltpu
from jax.experimental.pallas import tpu_sc as plsc

EMBED_DIM = 32
NUM_CORES = 2
NUM_SUBCORES = 16
NUM_WORKERS = NUM_CORES * NUM_SUBCORES
CHUNK = 1280


@functools.partial(jax.jit, static_argnames=("b_per_w", "n_chunks"))
def _emb_lookup(x_flat, embed, *, b_per_w, n_chunks):
    B = x_flat.shape[0]
    assert n_chunks % 2 == 0 and n_chunks >= 4
    mesh = plsc.VectorSubcoreMesh(core_axis_name="c", subcore_axis_name="s")

    @functools.partial(
        pl.kernel,
        mesh=mesh,
        out_type=jax.ShapeDtypeStruct((B, EMBED_DIM), jnp.float32),
        scratch_types=[
            pltpu.VMEM((b_per_w,), jnp.int32),
            pltpu.VMEM((2, CHUNK, EMBED_DIM), jnp.float32),
            pltpu.SemaphoreType.DMA,
            pltpu.SemaphoreType.DMA,
            pltpu.SemaphoreType.DMA,
            pltpu.SemaphoreType.DMA,
        ],
        compiler_params=pltpu.CompilerParams(use_tc_tiling_on_sc=False),
    )
    def emb_kernel(idx_hbm, table_hbm, out_hbm, idx_v, rows_v, gsem0, gsem1,
                   ssem0, ssem1):
        wid = lax.axis_index("s") * NUM_CORES + lax.axis_index("c")
        base = wid * b_per_w
        gsems = (gsem0, gsem1)
        ssems = (ssem0, ssem1)

        def gather_desc(g, slot):
            idx_c = idx_v.at[pl.ds(g * CHUNK, CHUNK)]
            return pltpu.make_async_copy(table_hbm.at[idx_c],
                                         rows_v.at[slot], gsems[slot])

        def store_desc(g, slot):
            dst = out_hbm.at[pl.ds(base + g * CHUNK, CHUNK)]
            return pltpu.make_async_copy(rows_v.at[slot], dst, ssems[slot])

        pltpu.sync_copy(idx_hbm.at[pl.ds(base, b_per_w)], idx_v)

        gather_desc(0, 0).start()
        gather_desc(1, 1).start()
        gather_desc(0, 0).wait()
        store_desc(0, 0).start()
        gather_desc(1, 1).wait()
        store_desc(1, 1).start()

        def body(gg, carry):
            for b in range(2):
                g = 2 * gg + b
                store_desc(g - 2, b).wait()
                gather_desc(g, b).start()
            for b in range(2):
                g = 2 * gg + b
                gather_desc(g, b).wait()
                store_desc(g, b).start()
            return carry

        lax.fori_loop(1, n_chunks // 2, body, 0)

        store_desc(n_chunks - 2, 0).wait()
        store_desc(n_chunks - 1, 1).wait()

    return emb_kernel(x_flat, embed)


def kernel(x, embed):
    B = x.shape[0] * x.shape[1]
    b_per_w = B // NUM_WORKERS
    n_chunks = b_per_w // CHUNK
    assert b_per_w % CHUNK == 0
    x_flat = x.reshape(B)
    out = _emb_lookup(x_flat, embed, b_per_w=b_per_w, n_chunks=n_chunks)
    return out.reshape(x.shape[0], x.shape[1], EMBED_DIM)


# speedup vs baseline: 1.8003x; 1.6213x over previous
import functools

import jax
import jax.numpy as jnp
from jax import lax
from jax.experimental import pallas as pl
from jax.experimental.pallas import tpu as pltpu
from jax.experimental.pallas import tpu_sc as plsc

EMBED_DIM = 32
NUM_CORES = 2
NUM_SUBCORES = 16
NUM_WORKERS = NUM_CORES * NUM_SUBCORES
XR_CHUNK = 16


@functools.partial(jax.jit, static_argnames=("n_rows", "n_cols"))
def _emb_lookup(x_flat, embed, *, n_rows, n_cols):
    B = x_flat.shape[0]
    rows_per_w = n_rows // NUM_WORKERS
    b_per_w = rows_per_w * n_cols
    n_chunks = rows_per_w // XR_CHUNK
    chunk_f = XR_CHUNK * n_cols
    assert rows_per_w % XR_CHUNK == 0 and n_chunks % 2 == 0
    mesh = plsc.VectorSubcoreMesh(core_axis_name="c", subcore_axis_name="s")

    @functools.partial(
        pl.kernel,
        mesh=mesh,
        out_type=jax.ShapeDtypeStruct((n_rows, n_cols, EMBED_DIM),
                                      jnp.float32),
        scratch_types=[
            pltpu.VMEM((b_per_w,), jnp.int32),
            pltpu.VMEM((2, chunk_f, EMBED_DIM), jnp.float32),
            pltpu.SemaphoreType.DMA,
            pltpu.SemaphoreType.DMA,
            pltpu.SemaphoreType.DMA,
            pltpu.SemaphoreType.DMA,
        ],
        compiler_params=pltpu.CompilerParams(use_tc_tiling_on_sc=False),
    )
    def emb_kernel(idx_hbm, table_hbm, out_hbm, idx_v, rows_v, gsem0, gsem1,
                   ssem0, ssem1):
        wid = lax.axis_index("s") * NUM_CORES + lax.axis_index("c")
        base = wid * b_per_w
        xrow0 = wid * rows_per_w
        gsems = (gsem0, gsem1)
        ssems = (ssem0, ssem1)

        def gather_desc(g, slot):
            idx_c = idx_v.at[pl.ds(g * chunk_f, chunk_f)]
            return pltpu.make_async_copy(table_hbm.at[idx_c],
                                         rows_v.at[slot], gsems[slot])

        def store_descs(g, slot):
            out = []
            for r in range(XR_CHUNK):
                src = rows_v.at[slot, pl.ds(r * n_cols, n_cols)]
                dst = out_hbm.at[xrow0 + g * XR_CHUNK + r]
                out.append(pltpu.make_async_copy(src, dst, ssems[slot]))
            return out

        def store_start(g, slot):
            for d in store_descs(g, slot):
                d.start()

        def store_wait(g, slot):
            for d in store_descs(g, slot):
                d.wait()

        pltpu.sync_copy(idx_hbm.at[pl.ds(base, b_per_w)], idx_v)

        gather_desc(0, 0).start()
        gather_desc(1, 1).start()
        gather_desc(0, 0).wait()
        store_start(0, 0)
        gather_desc(1, 1).wait()
        store_start(1, 1)

        def body(gg, carry):
            for b in range(2):
                g = 2 * gg + b
                store_wait(g - 2, b)
                gather_desc(g, b).start()
            for b in range(2):
                g = 2 * gg + b
                gather_desc(g, b).wait()
                store_start(g, b)
            return carry

        lax.fori_loop(1, n_chunks // 2, body, 0)

        store_wait(n_chunks - 2, 0)
        store_wait(n_chunks - 1, 1)

    return emb_kernel(x_flat, embed)


def kernel(x, embed):
    n_rows, n_cols = x.shape
    x_flat = x.reshape(n_rows * n_cols)
    return _emb_lookup(x_flat, embed, n_rows=n_rows, n_cols=n_cols)


# speedup vs baseline: 1.8067x; 1.0035x over previous
import functools

import jax
import jax.numpy as jnp
from jax import lax
from jax.experimental import pallas as pl
from jax.experimental.pallas import tpu as pltpu
from jax.experimental.pallas import tpu_sc as plsc

EMBED_DIM = 32
NUM_CORES = 2
NUM_SUBCORES = 16
NUM_WORKERS = NUM_CORES * NUM_SUBCORES
XR_CHUNK = 16


@functools.partial(jax.jit, static_argnames=("n_rows", "n_cols"))
def _emb_lookup(x_flat, embed, *, n_rows, n_cols):
    B = x_flat.shape[0]
    rows_per_w = n_rows // NUM_WORKERS
    b_per_w = rows_per_w * n_cols
    n_chunks = rows_per_w // XR_CHUNK
    chunk_f = XR_CHUNK * n_cols
    assert rows_per_w % XR_CHUNK == 0 and n_chunks % 2 == 0
    mesh = plsc.VectorSubcoreMesh(core_axis_name="c", subcore_axis_name="s")

    @functools.partial(
        pl.kernel,
        mesh=mesh,
        out_type=jax.ShapeDtypeStruct((n_rows, n_cols, EMBED_DIM),
                                      jnp.float32),
        scratch_types=[
            pltpu.VMEM((b_per_w,), jnp.int32),
            pltpu.VMEM((2, chunk_f, EMBED_DIM), jnp.float32),
            pltpu.SemaphoreType.DMA,
            pltpu.SemaphoreType.DMA,
            pltpu.SemaphoreType.DMA,
            pltpu.SemaphoreType.DMA,
        ],
        compiler_params=pltpu.CompilerParams(use_tc_tiling_on_sc=False),
    )
    def emb_kernel(idx_hbm, table_hbm, out_hbm, idx_v, rows_v, gsem0, gsem1,
                   ssem0, ssem1):
        wid = lax.axis_index("s") * NUM_CORES + lax.axis_index("c")
        base = wid * b_per_w
        xrow0 = wid * rows_per_w
        gsems = (gsem0, gsem1)
        ssems = (ssem0, ssem1)

        def gather_desc(g, slot):
            idx_c = idx_v.at[pl.ds(g * chunk_f, chunk_f)]
            return pltpu.make_async_copy(table_hbm.at[idx_c],
                                         rows_v.at[slot], gsems[slot])

        def store_descs(g, slot):
            out = []
            for r in range(XR_CHUNK):
                src = rows_v.at[slot, pl.ds(r * n_cols, n_cols)]
                dst = out_hbm.at[xrow0 + g * XR_CHUNK + r]
                out.append(pltpu.make_async_copy(src, dst, ssems[slot]))
            return out

        def store_start(g, slot):
            for d in store_descs(g, slot):
                d.start()

        def store_wait(g, slot):
            for d in store_descs(g, slot):
                d.wait()

        pltpu.sync_copy(idx_hbm.at[pl.ds(base, b_per_w)], idx_v)

        gather_desc(0, 0).start()
        gather_desc(1, 1).start()
        gather_desc(0, 0).wait()
        store_start(0, 0)
        gather_desc(1, 1).wait()
        store_start(1, 1)

        def body(gg, carry):
            for b in range(2):
                g = 2 * gg + b
                store_wait(g - 2, b)
                gather_desc(g, b).start()
            for b in range(2):
                g = 2 * gg + b
                gather_desc(g, b).wait()
                store_start(g, b)
            return carry

        lax.fori_loop(1, n_chunks // 2, body, 0)

        store_wait(n_chunks - 2, 0)
        store_wait(n_chunks - 1, 1)

    return emb_kernel(x_flat, embed)


def kernel(x, embed):
    n_rows, n_cols = x.shape
    x_flat = jnp.maximum(x.reshape(n_rows * n_cols), 0)
    table = embed + jnp.float32(0.0)
    return _emb_lookup(x_flat, table, n_rows=n_rows, n_cols=n_cols)
